# baseline (speedup 1.0000x reference)
import jax
import jax.numpy as jnp
from jax.experimental import pallas as pl

_MAX_H = 64
_MAX_W = 64
_S = _MAX_H * _MAX_W


def _add_body(x_ref, pe_ref, o_ref):
    o_ref[...] = x_ref[...] + pe_ref[...]


def kernel(x_flat, height, width, pos_embed):
    B, S = x_flat.shape
    offset = (jnp.asarray(height, jnp.int32) - _MAX_H) + (
        jnp.asarray(width, jnp.int32) - _MAX_W
    )
    idx = jnp.clip(jnp.arange(S, dtype=jnp.int32) + offset, 0, S - 1)
    pe = (jnp.take(pos_embed, idx) * 0.1).reshape(1, S)

    BR = 256
    out = pl.pallas_call(
        _add_body,
        grid=(B // BR,),
        in_specs=[
            pl.BlockSpec((BR, S), lambda i: (i, 0)),
            pl.BlockSpec((1, S), lambda i: (0, 0)),
        ],
        out_specs=pl.BlockSpec((BR, S), lambda i: (i, 0)),
        out_shape=jax.ShapeDtypeStruct((B, S), jnp.float32),
    )(x_flat, pe)
    return out


# speedup vs baseline: 1.0209x; 1.0209x over previous
import jax
import jax.numpy as jnp
from jax.experimental import pallas as pl

_MAX_H = 64
_MAX_W = 64
_S = _MAX_H * _MAX_W


def _add_body(x_ref, pe_ref, o_ref):
    o_ref[...] = x_ref[...] + pe_ref[...]


def kernel(x_flat, height, width, pos_embed):
    B, S = x_flat.shape
    offset = (jnp.asarray(height, jnp.int32) - _MAX_H) + (
        jnp.asarray(width, jnp.int32) - _MAX_W
    )
    idx = jnp.clip(jnp.arange(S, dtype=jnp.int32) + offset, 0, S - 1)
    pe = (jnp.take(pos_embed, idx) * 0.1).reshape(1, S)

    BR = 512
    out = pl.pallas_call(
        _add_body,
        grid=(B // BR,),
        in_specs=[
            pl.BlockSpec((BR, S), lambda i: (i, 0)),
            pl.BlockSpec((1, S), lambda i: (0, 0)),
        ],
        out_specs=pl.BlockSpec((BR, S), lambda i: (i, 0)),
        out_shape=jax.ShapeDtypeStruct((B, S), jnp.float32),
    )(x_flat, pe)
    return out
